# baseline (speedup 1.0000x reference)
import functools
import math

import jax
import jax.numpy as jnp
from jax.experimental import pallas as pl

NUM_NODES = 1000000
MEM_DIM = 128
TIME_DIM = 128
RAW_DIM = 128


def _enc_body(t_ref, lu_ref, w_ref, b_ref, out_ref):
    rel = t_ref[...] - lu_ref[...]
    out_ref[...] = jnp.cos(rel * w_ref[...] + b_ref[...])


def _time_encode(t, last_update, w_t, b_t, blk=2048):
    e = t.shape[0]
    grid = e // blk
    return pl.pallas_call(
        _enc_body,
        grid=(grid,),
        in_specs=[
            pl.BlockSpec((blk, 1), lambda i: (i, 0)),
            pl.BlockSpec((blk, 1), lambda i: (i, 0)),
            pl.BlockSpec((1, TIME_DIM), lambda i: (0, 0)),
            pl.BlockSpec((1, TIME_DIM), lambda i: (0, 0)),
        ],
        out_specs=pl.BlockSpec((blk, TIME_DIM), lambda i: (i, 0)),
        out_shape=jax.ShapeDtypeStruct((e, TIME_DIM), jnp.float32),
    )(t[:, None], last_update[:, None], w_t, b_t[None, :])


def _cell_body(pm_ref, aggr_ref, cnt_ref, wq_ref, wk_ref, wv_ref, wo_ref, out_ref):
    pm = pm_ref[...]
    aggr = aggr_ref[...] / jnp.clip(cnt_ref[...], 1.0, None)
    q = jnp.dot(pm, wq_ref[...], preferred_element_type=jnp.float32)
    k = jnp.dot(aggr, wk_ref[...], preferred_element_type=jnp.float32)
    v = jnp.dot(aggr, wv_ref[...], preferred_element_type=jnp.float32)
    score = jax.nn.sigmoid(
        jnp.sum(q * k, axis=-1, keepdims=True) / math.sqrt(float(MEM_DIM)))
    out_ref[...] = jnp.tanh(
        jnp.dot(pm + score * v, wo_ref[...], preferred_element_type=jnp.float32))


def _attention_cell(nid_prev_memory, aggr, counts, Wq, Wk, Wv, Wo, blk=1024):
    b = nid_prev_memory.shape[0]
    msg_out = aggr.shape[1]
    return pl.pallas_call(
        _cell_body,
        grid=(b // blk,),
        in_specs=[
            pl.BlockSpec((blk, MEM_DIM), lambda i: (i, 0)),
            pl.BlockSpec((blk, msg_out), lambda i: (i, 0)),
            pl.BlockSpec((blk, 1), lambda i: (i, 0)),
            pl.BlockSpec((MEM_DIM, MEM_DIM), lambda i: (0, 0)),
            pl.BlockSpec((msg_out, MEM_DIM), lambda i: (0, 0)),
            pl.BlockSpec((msg_out, MEM_DIM), lambda i: (0, 0)),
            pl.BlockSpec((MEM_DIM, MEM_DIM), lambda i: (0, 0)),
        ],
        out_specs=pl.BlockSpec((blk, MEM_DIM), lambda i: (i, 0)),
        out_shape=jax.ShapeDtypeStruct((b, MEM_DIM), jnp.float32),
    )(nid_prev_memory, aggr, counts[:, None], Wq, Wk, Wv, Wo)


def kernel(n_id, nid_prev_memory, src_s, dst_s, t_s, raw_msg_s, src_d, dst_d,
           t_d, raw_msg_d, src_prev_memory, dst_prev_memory, last_update,
           assoc, w_t, b_t, Wq, Wk, Wv, Wo):
    bn = n_id.shape[0]

    assoc = assoc.at[n_id].set(jnp.arange(bn, dtype=assoc.dtype))

    enc_s = _time_encode(t_s, last_update, w_t, b_t)
    enc_d = _time_encode(t_d, last_update, w_t, b_t)

    msg_s = jnp.concatenate([src_prev_memory, dst_prev_memory, raw_msg_s, enc_s], axis=-1)
    msg_d = jnp.concatenate([src_prev_memory, dst_prev_memory, raw_msg_d, enc_d], axis=-1)

    idx = jnp.concatenate([src_s, src_d], axis=0)
    msg = jnp.concatenate([msg_s, msg_d], axis=0)
    t = jnp.concatenate([t_s, t_d], axis=0)

    local_idx = assoc[idx]
    counts = jax.ops.segment_sum(jnp.ones_like(t), local_idx, num_segments=bn)
    aggr = jax.ops.segment_sum(msg, local_idx, num_segments=bn)

    updated_memory = _attention_cell(nid_prev_memory, aggr, counts, Wq, Wk, Wv, Wo)

    lu = jax.ops.segment_max(t, idx, num_segments=NUM_NODES)
    lu = jnp.where(jnp.isfinite(lu), lu, 0.0)
    updated_last_update = lu[n_id]

    return updated_memory, updated_last_update, assoc
